# baseline (speedup 1.0000x reference)
import jax
import jax.numpy as jnp
from jax import lax
from jax.experimental import pallas as pl
from jax.experimental.pallas import tpu as pltpu
from jax._src.pallas.mosaic.primitives import make_async_copy as _make_async_copy

EPS = 1e-5


def _make_kernel(nk, ch, nb, hw, lc):
    c8 = ch // 8
    kc = hw // lc

    def _norm_kernel(*refs):
        xs = refs[:nk]
        m_hbm, g1_hbm, bt_hbm, o_ref = refs[nk:nk + 4]
        m_s, g1_s, bt_s, sem_m, sem_g, sem_b = refs[nk + 4:nk + 10]
        n0 = pl.program_id(0)

        @pl.when(n0 == 0)
        def _():
            cm = _make_async_copy(m_hbm, m_s, sem_m)
            cg = _make_async_copy(g1_hbm, g1_s, sem_g)
            cb = _make_async_copy(bt_hbm, bt_s, sem_b)
            cm.start()
            cg.start()
            cb.start()
            cm.wait()
            cg.wait()
            cb.wait()

        invs = []
        for b in range(nb):
            mrow = m_s[pl.ds(n0 * nb + b, 1), 0, :]
            num = jnp.sum(mrow, axis=-1, keepdims=True)
            invs.append(1.0 / (num + EPS))

        for i in range(nk):
            x_r = xs[i]

            def _tile(q, _, x_r=x_r, i=i):
                bb = q // c8
                tt = q % c8
                bd = pl.ds(bb, 1)
                mb = pl.ds(n0 * nb + bb, 1)
                rows = pl.ds(tt * 8, 8)
                inv = invs[0]
                for b in range(1, nb):
                    inv = jnp.where(bb == b, invs[b], inv)
                xc0 = x_r[bd, rows, 0:lc].reshape(8, lc)
                mc0 = m_s[mb, :, 0:lc].reshape(8, lc)
                p = mc0 * xc0
                acc1 = p
                acc2 = p * xc0
                for k in range(1, kc):
                    xc = x_r[bd, rows, k * lc:(k + 1) * lc].reshape(8, lc)
                    mc = m_s[mb, :, k * lc:(k + 1) * lc].reshape(8, lc)
                    p = mc * xc
                    acc1 = acc1 + p
                    acc2 = acc2 + p * xc
                s1 = jnp.sum(acc1, axis=-1, keepdims=True)
                s2 = jnp.sum(acc2, axis=-1, keepdims=True)
                mu = s1 * inv
                var = jnp.maximum(s2 * inv - mu * mu, 0.0)
                gt = pl.ds(i * c8 + tt, 1)
                a = lax.rsqrt(var + EPS) * g1_s[0, gt, :, :].reshape(8, 1)
                a1 = a - 1.0
                bc = bt_s[0, gt, :, :].reshape(8, 1) - mu * a
                orows = pl.ds(i * ch + tt * 8, 8)
                for k in range(kc):
                    xc = x_r[bd, rows, k * lc:(k + 1) * lc].reshape(8, lc)
                    mc = m_s[mb, :, k * lc:(k + 1) * lc].reshape(8, lc)
                    o_ref[bd, orows, k * lc:(k + 1) * lc] = (
                        xc + mc * (xc * a1 + bc)).reshape(1, 8, lc)
                return _

            lax.fori_loop(0, nb * c8, _tile, 0, unroll=False)
    return _norm_kernel


def kernel(x, mask, gamma, beta):
    N, C, H, W = x.shape
    mh, mw = mask.shape[2], mask.shape[3]
    fh, fw = H // mh, W // mw
    HW = H * W

    m = jnp.broadcast_to(
        mask.reshape(N, 1, mh, 1, mw, 1), (N, 8, mh, fh, mw, fw)
    ).reshape(N, 8, HW).astype(jnp.float32)

    x_f = x.reshape(N, C, HW)
    g1 = (1.0 + gamma).astype(jnp.float32).reshape(1, C // 8, 8, 1)
    bt = beta.astype(jnp.float32).reshape(1, C // 8, 8, 1)

    B = 2 if N % 2 == 0 else 1
    K = 4 if C % 32 == 0 else 1
    Ch = C // K
    LC = next(lc for lc in (512, 256, 128, HW) if HW % lc == 0)
    grid = (N // B,)

    out = pl.pallas_call(
        _make_kernel(K, Ch, B, HW, LC),
        out_shape=jax.ShapeDtypeStruct((N, C, HW), x.dtype),
        grid=grid,
        in_specs=(
            [pl.BlockSpec((B, Ch, HW), lambda n, i=i: (n, i, 0))
             for i in range(K)]
            + [pl.BlockSpec(memory_space=pl.ANY),
               pl.BlockSpec(memory_space=pl.ANY),
               pl.BlockSpec(memory_space=pl.ANY)]
        ),
        out_specs=pl.BlockSpec((B, C, HW), lambda n: (n, 0, 0)),
        scratch_shapes=[
            pltpu.VMEM((N, 8, HW), jnp.float32),
            pltpu.VMEM((1, C // 8, 8, 1), jnp.float32),
            pltpu.VMEM((1, C // 8, 8, 1), jnp.float32),
            pltpu.SemaphoreType.DMA,
            pltpu.SemaphoreType.DMA,
            pltpu.SemaphoreType.DMA,
        ],
        compiler_params=pltpu.CompilerParams(
            dimension_semantics=("arbitrary",),
            vmem_limit_bytes=64 * 1024 * 1024,
        ),
    )(*([x_f] * K + [m, g1, bt]))
    return out.reshape(N, C, H, W)


# speedup vs baseline: 1.4872x; 1.4872x over previous
import jax
import jax.numpy as jnp
from jax import lax
from jax.experimental import pallas as pl
from jax.experimental.pallas import tpu as pltpu

EPS = 1e-5


def _norm_kernel(x_ref, m_ref, g1_ref, bt_ref, o_ref):
    m = m_ref[...]
    num = jnp.sum(m, axis=-1, keepdims=True)
    inv = 1.0 / (num + EPS)

    x = x_ref[...]
    r = m * x
    s1 = jnp.sum(r, axis=-1, keepdims=True)
    s2 = jnp.sum(r * r, axis=-1, keepdims=True)
    mu = s1 * inv
    var = jnp.maximum(s2 * inv - mu * mu, 0.0)
    a = lax.rsqrt(var + EPS) * g1_ref[...]
    b = bt_ref[...] - mu * a
    o_ref[...] = jnp.where(m > 0.0, x * a + b, x)


def kernel(x, mask, gamma, beta):
    N, C, H, W = x.shape
    mh, mw = mask.shape[2], mask.shape[3]
    fh, fw = H // mh, W // mw
    HW = H * W

    m = jnp.broadcast_to(
        mask.reshape(N, 1, mh, 1, mw, 1), (N, 1, mh, fh, mw, fw)
    ).reshape(N, 1, HW).astype(jnp.float32)

    x_f = x.reshape(N, C, HW)
    g1 = (1.0 + gamma).astype(jnp.float32).reshape(1, C, 1)
    bt = beta.astype(jnp.float32).reshape(1, C, 1)

    B = 2 if N % 2 == 0 else 1
    grid = (N // B,)

    out = pl.pallas_call(
        _norm_kernel,
        out_shape=jax.ShapeDtypeStruct((N, C, HW), x.dtype),
        grid=grid,
        in_specs=[
            pl.BlockSpec((B, C, HW), lambda n: (n, 0, 0)),
            pl.BlockSpec((B, 1, HW), lambda n: (n, 0, 0)),
            pl.BlockSpec((1, C, 1), lambda n: (0, 0, 0)),
            pl.BlockSpec((1, C, 1), lambda n: (0, 0, 0)),
        ],
        out_specs=pl.BlockSpec((B, C, HW), lambda n: (n, 0, 0)),
        compiler_params=pltpu.CompilerParams(
            dimension_semantics=("parallel",),
            vmem_limit_bytes=64 * 1024 * 1024,
        ),
    )(x_f, m, g1, bt)
    return out.reshape(N, C, H, W)
